# Initial kernel scaffold; baseline (speedup 1.0000x reference)
import jax
import jax.numpy as jnp
from jax.experimental import pallas as pl


def kernel(x_question, y_knowledge, seq_q, qkc_q, qkc_kc, kcs_src, kcs_dst, qs_q, qs_stu, W_kc, W_s, gamma0, beta0, gamma1, beta1):
    raise NotImplementedError("write your pallas kernel here")


# speedup vs baseline: 1.0356x; 1.0356x over previous
import functools

import jax
import jax.numpy as jnp
from jax.experimental import pallas as pl
from jax.experimental.pallas import tpu as pltpu


def _ln0_body(x_ref, g_ref, b_ref, o_ref):
    x = x_ref[...]
    mu = jnp.mean(x, axis=-1, keepdims=True)
    var = jnp.mean((x - mu) * (x - mu), axis=-1, keepdims=True)
    o_ref[...] = (x - mu) * jax.lax.rsqrt(var + 1e-5) * g_ref[...] + b_ref[...]


def _gather_head_body(qi_ref, bi_ref, hq_ref, hs_ref, wkc_ref, ws_ref,
                      g_ref, be_ref, oq_ref, os_ref):
    g = g_ref[...]
    be = be_ref[...]
    b = bi_ref[pl.program_id(0)]

    def head(h_ref, w):
        x = h_ref[0, pl.ds(b, 1), :]
        y = jnp.maximum(jnp.dot(x, w, preferred_element_type=jnp.float32), 0.0)
        mu = jnp.mean(y, axis=-1, keepdims=True)
        var = jnp.mean((y - mu) * (y - mu), axis=-1, keepdims=True)
        return (y - mu) * jax.lax.rsqrt(var + 1e-5) * g + be

    oq_ref[0] = head(hq_ref, wkc_ref[...])
    os_ref[0] = head(hs_ref, ws_ref[...])


def kernel(x_question, y_knowledge, seq_q, qkc_q, qkc_kc, kcs_src, kcs_dst,
           qs_q, qs_stu, W_kc, W_s, gamma0, beta0, gamma1, beta1):
    numq, bs, d = x_question.shape
    num_kc = y_knowledge.shape[0]
    num_stu = 5001
    seq_len = seq_q.shape[1]

    rowb = 128
    grid0 = (numq + rowb - 1) // rowb
    new_feature = pl.pallas_call(
        _ln0_body,
        grid=(grid0,),
        in_specs=[
            pl.BlockSpec((rowb, bs, d), lambda i: (i, 0, 0)),
            pl.BlockSpec((1, 1, d), lambda i: (0, 0, 0)),
            pl.BlockSpec((1, 1, d), lambda i: (0, 0, 0)),
        ],
        out_specs=pl.BlockSpec((rowb, bs, d), lambda i: (i, 0, 0)),
        out_shape=jax.ShapeDtypeStruct((numq, bs, d), jnp.float32),
    )(x_question, gamma0.reshape(1, 1, d), beta0.reshape(1, 1, d))

    m_kc = jax.ops.segment_sum(new_feature[qkc_q], qkc_kc, num_segments=num_kc)
    m_kc = m_kc + jax.ops.segment_sum(m_kc[kcs_src], kcs_dst, num_segments=num_kc)
    h_qkc = jax.ops.segment_sum(m_kc[qkc_kc], qkc_q, num_segments=numq)
    m_stu = jax.ops.segment_sum(new_feature[qs_q], qs_stu, num_segments=num_stu)
    h_qs = jax.ops.segment_sum(m_stu[qs_stu], qs_q, num_segments=numq)

    n = bs * seq_len
    q_idx = seq_q.reshape(-1).astype(jnp.int32)
    b_idx = jnp.repeat(jnp.arange(bs, dtype=jnp.int32), seq_len)

    grid_spec = pltpu.PrefetchScalarGridSpec(
        num_scalar_prefetch=2,
        grid=(n,),
        in_specs=[
            pl.BlockSpec((1, bs, d), lambda i, qi, bi: (qi[i], 0, 0)),
            pl.BlockSpec((1, bs, d), lambda i, qi, bi: (qi[i], 0, 0)),
            pl.BlockSpec((d, d), lambda i, qi, bi: (0, 0)),
            pl.BlockSpec((d, d), lambda i, qi, bi: (0, 0)),
            pl.BlockSpec((1, d), lambda i, qi, bi: (0, 0)),
            pl.BlockSpec((1, d), lambda i, qi, bi: (0, 0)),
        ],
        out_specs=[
            pl.BlockSpec((1, 1, d), lambda i, qi, bi: (i, 0, 0)),
            pl.BlockSpec((1, 1, d), lambda i, qi, bi: (i, 0, 0)),
        ],
    )
    out_q, out_s = pl.pallas_call(
        _gather_head_body,
        grid_spec=grid_spec,
        out_shape=[
            jax.ShapeDtypeStruct((n, 1, d), jnp.float32),
            jax.ShapeDtypeStruct((n, 1, d), jnp.float32),
        ],
    )(q_idx, b_idx, h_qkc, h_qs, W_kc, W_s,
      gamma1.reshape(1, d), beta1.reshape(1, d))

    return (out_q.reshape(bs, seq_len, d), out_s.reshape(bs, seq_len, d))
